# baseline (speedup 1.0000x reference)
import jax
import jax.numpy as jnp
from jax import lax
from jax.experimental import pallas as pl
from jax.experimental.pallas import tpu as pltpu
from jax.experimental.pallas import tpu_sc as plsc

N = 10000
E = 320000
R = 8
F = 128
H = 128
O = 64
RN = R * N
PAD = 81920
NC = 2
NS = 16
NW = NC * NS
ET = E // NW
BT = 80
CH = 2000
ROWS = CH // BT
NCH = ET // CH
BN = 2000
NB = N // BN
ZCH = PAD // NS

_sc_mesh = plsc.VectorSubcoreMesh(
    core_axis_name="c", subcore_axis_name="s", num_cores=NC, num_subcores=NS)


ETP = E // NS
NCHP = ETP // CH


def _prep_body(esrc_ref, edst_ref, et_ref, z_ref, lut_ref,
               gidx_ref, kdst_ref, norms_ref,
               n_v, typ_v, k1_v, k2_v, ones_v, degf_v, degi_v, nrm_v,
               acc, sem_a, sem_b):
    c = lax.axis_index("c")
    s = lax.axis_index("s")

    def fill_ones(k, _):
        ones_v[pl.ds(k * 16, 16)] = jnp.ones((16,), jnp.float32)
        return 0
    lax.fori_loop(0, BT // 16, fill_ones, 0)

    pltpu.sync_copy(z_ref, acc.at[pl.ds(s * ZCH, ZCH)])
    plsc.subcore_barrier()

    def chunk(j, _):
        ebase = s * ETP + j * CH

        @pl.when(c == 0)
        def _():
            pltpu.sync_copy(esrc_ref.at[pl.ds(ebase, CH)], n_v)

        @pl.when(c == 1)
        def _():
            pltpu.sync_copy(edst_ref.at[pl.ds(ebase, CH)], n_v)
        pltpu.sync_copy(et_ref.at[pl.ds(ebase, CH)], typ_v)

        def vec(k, _):
            r = k // (BT // 16)
            q = (k % (BT // 16)) * 16
            k16 = typ_v[pl.ds(k * 16, 16)] * N + n_v[pl.ds(k * 16, 16)]
            k1_v[pl.ds(k * 16, 16)] = k16
            k2_v[r, pl.ds(q, 16)] = k16
            return 0
        lax.fori_loop(0, CH // 16, vec, 0)

        @pl.when(c == 0)
        def _():
            pltpu.sync_copy(k1_v, gidx_ref.at[pl.ds(ebase, CH)])

        @pl.when(c == 1)
        def _():
            pltpu.sync_copy(k1_v, kdst_ref.at[pl.ds(ebase, CH)])

        def scat_fire(r, _):
            pltpu.async_copy(ones_v, acc.at[k2_v.at[r]], sem_a, add=True)
            return 0
        lax.fori_loop(0, ROWS, scat_fire, 0)

        def scat_drain(r, _):
            pltpu.make_async_copy(ones_v, acc.at[k2_v.at[0]], sem_a).wait()
            return 0
        lax.fori_loop(0, ROWS, scat_drain, 0)
        return 0
    lax.fori_loop(0, NCHP, chunk, 0)
    plsc.subcore_barrier()

    pltpu.sync_copy(acc.at[pl.ds(s * ZCH, ZCH)], degf_v)

    def conv(k, _):
        degi_v[pl.ds(k * 16, 16)] = degf_v[pl.ds(k * 16, 16)].astype(
            jnp.int32)
        return 0
    lax.fori_loop(0, ZCH // 16, conv, 0)

    def nfire(g, _):
        pltpu.async_copy(lut_ref.at[degi_v.at[pl.ds(g * BT, BT)]],
                         nrm_v.at[pl.ds(g * BT, BT)], sem_b)
        return 0
    lax.fori_loop(0, ZCH // BT, nfire, 0)

    def ndrain(g, _):
        pltpu.make_async_copy(lut_ref.at[degi_v.at[pl.ds(0, BT)]],
                              nrm_v.at[pl.ds(0, BT)], sem_b).wait()
        return 0
    lax.fori_loop(0, ZCH // BT, ndrain, 0)
    pltpu.sync_copy(nrm_v, norms_ref.at[pl.ds(c * PAD + s * ZCH, ZCH)])


_edge_prep = pl.kernel(
    _prep_body,
    out_type=(
        jax.ShapeDtypeStruct((E,), jnp.int32),
        jax.ShapeDtypeStruct((E,), jnp.int32),
        jax.ShapeDtypeStruct((2 * PAD,), jnp.float32),
    ),
    mesh=_sc_mesh,
    scratch_types=[
        pltpu.VMEM((CH,), jnp.int32),
        pltpu.VMEM((CH,), jnp.int32),
        pltpu.VMEM((CH,), jnp.int32),
        pltpu.VMEM((ROWS, BT), jnp.int32),
        pltpu.VMEM((BT,), jnp.float32),
        pltpu.VMEM((ZCH,), jnp.float32),
        pltpu.VMEM((ZCH,), jnp.int32),
        pltpu.VMEM((ZCH,), jnp.float32),
        pltpu.VMEM_SHARED((PAD,), jnp.float32),
        pltpu.SemaphoreType.DMA,
        pltpu.SemaphoreType.DMA,
    ],
)


def _table1_body(x_ref, w_ref, o_ref):
    o_ref[...] = jnp.dot(x_ref[...], w_ref[0],
                         preferred_element_type=jnp.float32)


def _table1(x, W1):
    return pl.pallas_call(
        _table1_body,
        grid=(NB, R),
        in_specs=[
            pl.BlockSpec((BN, F), lambda i, r: (i, 0)),
            pl.BlockSpec((1, F, H), lambda i, r: (r, 0, 0)),
        ],
        out_specs=pl.BlockSpec((BN, H), lambda i, r: (r * NB + i, 0)),
        out_shape=jax.ShapeDtypeStruct((RN, H), jnp.float32),
    )(x, W1)


def _table2_body(hp_ref, b1_ref, w_ref, o_ref):
    mb = jnp.mean(b1_ref[...], axis=0)
    h = jnp.maximum((hp_ref[0] + hp_ref[1]) * (1.0 / R) + mb[None, :], 0.0)
    t = jnp.dot(h, w_ref[0], preferred_element_type=jnp.float32)
    o_ref[...] = jnp.concatenate([t, jnp.zeros((BN, H - O), jnp.float32)],
                                 axis=1)


def _table2(h1p, b1, W2):
    return pl.pallas_call(
        _table2_body,
        grid=(NB, R),
        in_specs=[
            pl.BlockSpec((2, BN, H), lambda i, r: (0, i, 0)),
            pl.BlockSpec((R, H), lambda i, r: (0, 0)),
            pl.BlockSpec((1, H, O), lambda i, r: (r, 0, 0)),
        ],
        out_specs=pl.BlockSpec((BN, H), lambda i, r: (r * NB + i, 0)),
        out_shape=jax.ShapeDtypeStruct((RN, H), jnp.float32),
    )(h1p, b1, W2)


def _run_chunk_pipeline(tab_ref, g1_v, b1_v, d2_v, rows_a, rows_b, acc,
                        sem_ga, sem_gb, sem_sa, sem_sb, nq):
    def gfire(r, buf, sem):
        pltpu.async_copy(tab_ref.at[g1_v.at[pl.ds(r * BT, BT)]], buf, sem)

    def gwait(buf, sem):
        pltpu.make_async_copy(
            tab_ref.at[g1_v.at[pl.ds(0, BT)]], buf, sem).wait()

    def swait(buf, sem):
        pltpu.make_async_copy(buf, acc.at[d2_v.at[0]], sem).wait()

    def batch(r, buf, gsem, ssem, obuf, ogsem, ossem):
        @pl.when(r > 0)
        def _():
            swait(obuf, ossem)

        @pl.when(r < ROWS - 1)
        def _():
            gfire(r + 1, obuf, ogsem)

        gwait(buf, gsem)

        def scale(g, _):
            c16 = b1_v[pl.ds(r * BT + g * 16, 16)]
            for j in range(16):
                cv = c16[j]
                e = g * 16 + j
                for q2 in range(nq):
                    buf[e, pl.ds(q2 * 16, 16)] = \
                        buf[e, pl.ds(q2 * 16, 16)] * cv
            return 0
        lax.fori_loop(0, BT // 16, scale, 0)
        pltpu.async_copy(buf, acc.at[d2_v.at[r]], ssem, add=True)

    gfire(0, rows_a, sem_ga)

    def step(r, _):
        even = lax.rem(r, 2) == 0

        @pl.when(even)
        def _():
            batch(r, rows_a, sem_ga, sem_sa, rows_b, sem_gb, sem_sb)

        @pl.when(jnp.logical_not(even))
        def _():
            batch(r, rows_b, sem_gb, sem_sb, rows_a, sem_ga, sem_sa)
        return 0
    lax.fori_loop(0, ROWS, step, 0)
    swait(rows_a, sem_sa)


def _pass1_body(tab_ref, gidx_ref, kdst_ref, edst_ref, nsrc_ref, ndst_ref,
                zr_ref,
                h1p_ref, b_ref,
                g1_v, d1_v, k1_v, b1_v, n1_v, d2_v, rows_a, rows_b, acc,
                sem_ga, sem_gb, sem_sa, sem_sb, sem_b):
    c = lax.axis_index("c")
    s = lax.axis_index("s")
    wid = s * NC + c
    nslc = N // 10
    ebase = wid * ET

    @pl.when(s < 10)
    def _zero():
        pltpu.sync_copy(zr_ref, acc.at[pl.ds(s * nslc, nslc)])
    plsc.subcore_barrier()

    def chunk(j, _):
        cbase = ebase + j * CH
        pltpu.sync_copy(gidx_ref.at[pl.ds(cbase, CH)], g1_v)
        pltpu.sync_copy(edst_ref.at[pl.ds(cbase, CH)], d1_v)
        pltpu.sync_copy(kdst_ref.at[pl.ds(cbase, CH)], k1_v)

        def repack(k, _):
            r = k // (BT // 16)
            q = (k % (BT // 16)) * 16
            d2_v[r, pl.ds(q, 16)] = d1_v[pl.ds(k * 16, 16)]
            return 0
        lax.fori_loop(0, CH // 16, repack, 0)

        def bfire(r, _):
            pltpu.async_copy(ndst_ref.at[k1_v.at[pl.ds(r * BT, BT)]],
                             b1_v.at[pl.ds(r * BT, BT)], sem_b)
            pltpu.async_copy(nsrc_ref.at[g1_v.at[pl.ds(r * BT, BT)]],
                             n1_v.at[pl.ds(r * BT, BT)], sem_b)
            return 0
        lax.fori_loop(0, ROWS, bfire, 0)

        def bdrain(r, _):
            pltpu.make_async_copy(ndst_ref.at[k1_v.at[pl.ds(0, BT)]],
                                  b1_v.at[pl.ds(0, BT)], sem_b).wait()
            pltpu.make_async_copy(ndst_ref.at[k1_v.at[pl.ds(0, BT)]],
                                  n1_v.at[pl.ds(0, BT)], sem_b).wait()
            return 0
        lax.fori_loop(0, ROWS, bdrain, 0)

        def combine(k, _):
            b1_v[pl.ds(k * 16, 16)] = (b1_v[pl.ds(k * 16, 16)]
                                       * n1_v[pl.ds(k * 16, 16)])
            return 0
        lax.fori_loop(0, CH // 16, combine, 0)
        pltpu.sync_copy(b1_v, b_ref.at[pl.ds(cbase, CH)])

        _run_chunk_pipeline(tab_ref, g1_v, b1_v, d2_v, rows_a, rows_b, acc,
                            sem_ga, sem_gb, sem_sa, sem_sb, H // 16)
        return 0
    lax.fori_loop(0, NCH, chunk, 0)
    plsc.subcore_barrier()

    @pl.when(s < 10)
    def _export():
        pltpu.sync_copy(acc.at[pl.ds(s * nslc, nslc)],
                        h1p_ref.at[c, pl.ds(s * nslc, nslc)])


_edge_pass1 = pl.kernel(
    _pass1_body,
    out_type=(
        jax.ShapeDtypeStruct((NC, N, H), jnp.float32),
        jax.ShapeDtypeStruct((E,), jnp.float32),
    ),
    mesh=_sc_mesh,
    scratch_types=[
        pltpu.VMEM((CH,), jnp.int32),
        pltpu.VMEM((CH,), jnp.int32),
        pltpu.VMEM((CH,), jnp.int32),
        pltpu.VMEM((CH,), jnp.float32),
        pltpu.VMEM((CH,), jnp.float32),
        pltpu.VMEM((ROWS, BT), jnp.int32),
        pltpu.VMEM((BT, H), jnp.float32),
        pltpu.VMEM((BT, H), jnp.float32),
        pltpu.VMEM_SHARED((N, H), jnp.float32),
        pltpu.SemaphoreType.DMA,
        pltpu.SemaphoreType.DMA,
        pltpu.SemaphoreType.DMA,
        pltpu.SemaphoreType.DMA,
        pltpu.SemaphoreType.DMA,
    ],
)


def _pass2_body(tab_ref, gidx_ref, edst_ref, b_ref, zr_ref,
                h2p_ref,
                g1_v, d1_v, b1_v, d2_v, rows_a, rows_b, acc,
                sem_ga, sem_gb, sem_sa, sem_sb):
    c = lax.axis_index("c")
    s = lax.axis_index("s")
    wid = s * NC + c
    nslc = N // 10
    ebase = wid * ET

    @pl.when(s < 10)
    def _zero():
        pltpu.sync_copy(zr_ref, acc.at[pl.ds(s * nslc, nslc)])
    plsc.subcore_barrier()

    def chunk(j, _):
        cbase = ebase + j * CH
        pltpu.sync_copy(gidx_ref.at[pl.ds(cbase, CH)], g1_v)
        pltpu.sync_copy(edst_ref.at[pl.ds(cbase, CH)], d1_v)
        pltpu.sync_copy(b_ref.at[pl.ds(cbase, CH)], b1_v)

        def repack(k, _):
            r = k // (BT // 16)
            q = (k % (BT // 16)) * 16
            d2_v[r, pl.ds(q, 16)] = d1_v[pl.ds(k * 16, 16)]
            return 0
        lax.fori_loop(0, CH // 16, repack, 0)

        _run_chunk_pipeline(tab_ref, g1_v, b1_v, d2_v, rows_a, rows_b, acc,
                            sem_ga, sem_gb, sem_sa, sem_sb, O // 16)
        return 0
    lax.fori_loop(0, NCH, chunk, 0)
    plsc.subcore_barrier()

    @pl.when(s < 10)
    def _export():
        pltpu.sync_copy(acc.at[pl.ds(s * nslc, nslc)],
                        h2p_ref.at[c, pl.ds(s * nslc, nslc)])


_edge_pass2 = pl.kernel(
    _pass2_body,
    out_type=jax.ShapeDtypeStruct((NC, N, H), jnp.float32),
    mesh=_sc_mesh,
    scratch_types=[
        pltpu.VMEM((CH,), jnp.int32),
        pltpu.VMEM((CH,), jnp.int32),
        pltpu.VMEM((CH,), jnp.float32),
        pltpu.VMEM((ROWS, BT), jnp.int32),
        pltpu.VMEM((BT, H), jnp.float32),
        pltpu.VMEM((BT, H), jnp.float32),
        pltpu.VMEM_SHARED((N, H), jnp.float32),
        pltpu.SemaphoreType.DMA,
        pltpu.SemaphoreType.DMA,
        pltpu.SemaphoreType.DMA,
        pltpu.SemaphoreType.DMA,
    ],
)


def _final_body(hp_ref, b2_ref, wf_ref, bf_ref, wb_ref, bb_ref, o_ref):
    mb = jnp.mean(b2_ref[...], axis=0)
    h = (hp_ref[0, :, :O] + hp_ref[1, :, :O]) * (1.0 / R) + mb[None, :]

    def lstm(w, bias):
        gates = jnp.dot(h, w, preferred_element_type=jnp.float32) + bias
        hh = O // 2
        i_ = gates[:, :hh]
        g_ = gates[:, 2 * hh:3 * hh]
        o_ = gates[:, 3 * hh:]
        cc = jax.nn.sigmoid(i_) * jnp.tanh(g_)
        return jax.nn.sigmoid(o_) * jnp.tanh(cc)

    o_ref[...] = jnp.concatenate(
        [lstm(wf_ref[...], bf_ref[...]), lstm(wb_ref[...], bb_ref[...])],
        axis=1)


def _final(h2p, b2, WihT_f, bias_f, WihT_b, bias_b):
    return pl.pallas_call(
        _final_body,
        grid=(NB,),
        in_specs=[
            pl.BlockSpec((2, BN, H), lambda i: (0, i, 0)),
            pl.BlockSpec((R, O), lambda i: (0, 0)),
            pl.BlockSpec((O, 2 * O), lambda i: (0, 0)),
            pl.BlockSpec((1, 2 * O), lambda i: (0, 0)),
            pl.BlockSpec((O, 2 * O), lambda i: (0, 0)),
            pl.BlockSpec((1, 2 * O), lambda i: (0, 0)),
        ],
        out_specs=pl.BlockSpec((BN, O), lambda i: (i, 0)),
        out_shape=jax.ShapeDtypeStruct((N, O), jnp.float32),
    )(h2p, b2, WihT_f, bias_f, WihT_b, bias_b)


def kernel(node_ids, edge_index, edge_type, entity_emb, W1, b1, W2, b2,
           Wih_f, bih_f, bhh_f, Wih_b, bih_b, bhh_b):
    x = jnp.take(entity_emb, node_ids, axis=0)
    z_deg = jnp.zeros((ZCH,), jnp.float32)
    zr = jnp.zeros((N // 10, H), jnp.float32)

    esrc = edge_index[0]
    edst = edge_index[1]
    lut = lax.rsqrt(jnp.maximum(jnp.arange(E + 1, dtype=jnp.float32), 1.0))
    gidx, kdst, norms = _edge_prep(esrc, edst, edge_type, z_deg, lut)
    nsrc1d = norms[:RN]
    ndst1d = norms[PAD:PAD + RN]

    table1 = _table1(x, W1)
    h1p, c_e = _edge_pass1(table1, gidx, kdst, edst, nsrc1d, ndst1d, zr)
    table2 = _table2(h1p, b1, W2)
    h2p = _edge_pass2(table2, gidx, edst, c_e, zr)

    bias_f = (bih_f + bhh_f).reshape(1, 2 * O)
    bias_b = (bih_b + bhh_b).reshape(1, 2 * O)
    return _final(h2p, b2, Wih_f.T, bias_f, Wih_b.T, bias_b)


# speedup vs baseline: 2.6177x; 2.6177x over previous
import jax
import jax.numpy as jnp
from jax import lax
from jax.experimental import pallas as pl
from jax.experimental.pallas import tpu as pltpu
from jax.experimental.pallas import tpu_sc as plsc

N = 10000
E = 320000
R = 8
F = 128
H = 128
O = 64
RN = R * N
PAD = 81920
NC = 2
NS = 16
NW = NC * NS
ET = E // NW
BT = 80
CH = 2000
ROWS = CH // BT
NCH = ET // CH
BN = 2000
NB = N // BN
ZCH = PAD // NS

_sc_mesh = plsc.VectorSubcoreMesh(
    core_axis_name="c", subcore_axis_name="s", num_cores=NC, num_subcores=NS)


def _prep_body(esrc_ref, edst_ref, et_ref, z_ref,
               gidx_ref, kdst_ref, degp_ref,
               src_v, typ_v, dst_v, g1_v, k1_v, g2_v, k2_v, ones_v,
               acc_s, acc_d, sem_a, sem_d):
    c = lax.axis_index("c")
    s = lax.axis_index("s")
    wid = s * NC + c

    def fill_ones(k, _):
        ones_v[pl.ds(k * 16, 16)] = jnp.ones((16,), jnp.float32)
        return 0
    lax.fori_loop(0, BT // 16, fill_ones, 0)

    pltpu.sync_copy(z_ref, acc_s.at[pl.ds(s * ZCH, ZCH)])
    pltpu.sync_copy(z_ref, acc_d.at[pl.ds(s * ZCH, ZCH)])
    plsc.subcore_barrier()

    def chunk(j, _):
        ebase = wid * ET + j * CH
        pltpu.sync_copy(esrc_ref.at[pl.ds(ebase, CH)], src_v)
        pltpu.sync_copy(edst_ref.at[pl.ds(ebase, CH)], dst_v)
        pltpu.sync_copy(et_ref.at[pl.ds(ebase, CH)], typ_v)

        def vec(k, _):
            r = k // (BT // 16)
            q = (k % (BT // 16)) * 16
            s16 = src_v[pl.ds(k * 16, 16)]
            d16 = dst_v[pl.ds(k * 16, 16)]
            t16 = typ_v[pl.ds(k * 16, 16)] * N
            g16 = t16 + s16
            k16 = t16 + d16
            g1_v[pl.ds(k * 16, 16)] = g16
            k1_v[pl.ds(k * 16, 16)] = k16
            g2_v[r, pl.ds(q, 16)] = g16
            k2_v[r, pl.ds(q, 16)] = k16
            return 0
        lax.fori_loop(0, CH // 16, vec, 0)

        pltpu.sync_copy(g1_v, gidx_ref.at[pl.ds(ebase, CH)])
        pltpu.sync_copy(k1_v, kdst_ref.at[pl.ds(ebase, CH)])

        def scat_fire(r, _):
            pltpu.async_copy(ones_v, acc_s.at[g2_v.at[r]], sem_a, add=True)
            pltpu.async_copy(ones_v, acc_d.at[k2_v.at[r]], sem_d, add=True)
            return 0
        lax.fori_loop(0, ROWS, scat_fire, 0)

        def scat_drain(r, _):
            pltpu.make_async_copy(ones_v, acc_s.at[g2_v.at[0]], sem_a).wait()
            pltpu.make_async_copy(ones_v, acc_d.at[k2_v.at[0]], sem_d).wait()
            return 0
        lax.fori_loop(0, ROWS, scat_drain, 0)
        return 0
    lax.fori_loop(0, NCH, chunk, 0)

    plsc.subcore_barrier()
    pltpu.sync_copy(acc_s.at[pl.ds(s * ZCH, ZCH)],
                    degp_ref.at[pl.ds(c * PAD + s * ZCH, ZCH)])
    pltpu.sync_copy(acc_d.at[pl.ds(s * ZCH, ZCH)],
                    degp_ref.at[pl.ds((2 + c) * PAD + s * ZCH, ZCH)])


_edge_prep = pl.kernel(
    _prep_body,
    out_type=(
        jax.ShapeDtypeStruct((E,), jnp.int32),
        jax.ShapeDtypeStruct((E,), jnp.int32),
        jax.ShapeDtypeStruct((4 * PAD,), jnp.float32),
    ),
    mesh=_sc_mesh,
    scratch_types=[
        pltpu.VMEM((CH,), jnp.int32),
        pltpu.VMEM((CH,), jnp.int32),
        pltpu.VMEM((CH,), jnp.int32),
        pltpu.VMEM((CH,), jnp.int32),
        pltpu.VMEM((CH,), jnp.int32),
        pltpu.VMEM((ROWS, BT), jnp.int32),
        pltpu.VMEM((ROWS, BT), jnp.int32),
        pltpu.VMEM((BT,), jnp.float32),
        pltpu.VMEM_SHARED((PAD,), jnp.float32),
        pltpu.VMEM_SHARED((PAD,), jnp.float32),
        pltpu.SemaphoreType.DMA,
        pltpu.SemaphoreType.DMA,
    ],
)


def _norm_body(degp_ref, out_ref):
    d = degp_ref[...]
    out_ref[0] = lax.rsqrt(jnp.maximum(d[0] + d[1], 1.0))
    out_ref[1] = lax.rsqrt(jnp.maximum(d[2] + d[3], 1.0))


def _norms(degp):
    return pl.pallas_call(
        _norm_body,
        out_shape=jax.ShapeDtypeStruct((2, PAD // 128, 128), jnp.float32),
    )(degp.reshape(4, PAD // 128, 128))


def _table1_body(x_ref, w_ref, o_ref):
    o_ref[...] = jnp.dot(x_ref[...], w_ref[0],
                         preferred_element_type=jnp.float32)


def _table1(x, W1):
    return pl.pallas_call(
        _table1_body,
        grid=(NB, R),
        in_specs=[
            pl.BlockSpec((BN, F), lambda i, r: (i, 0)),
            pl.BlockSpec((1, F, H), lambda i, r: (r, 0, 0)),
        ],
        out_specs=pl.BlockSpec((BN, H), lambda i, r: (r * NB + i, 0)),
        out_shape=jax.ShapeDtypeStruct((RN, H), jnp.float32),
    )(x, W1)


def _table2_body(hp_ref, b1_ref, w_ref, o_ref):
    mb = jnp.mean(b1_ref[...], axis=0)
    h = jnp.maximum((hp_ref[0] + hp_ref[1]) * (1.0 / R) + mb[None, :], 0.0)
    t = jnp.dot(h, w_ref[0], preferred_element_type=jnp.float32)
    o_ref[...] = jnp.concatenate([t, jnp.zeros((BN, H - O), jnp.float32)],
                                 axis=1)


def _table2(h1p, b1, W2):
    return pl.pallas_call(
        _table2_body,
        grid=(NB, R),
        in_specs=[
            pl.BlockSpec((2, BN, H), lambda i, r: (0, i, 0)),
            pl.BlockSpec((R, H), lambda i, r: (0, 0)),
            pl.BlockSpec((1, H, O), lambda i, r: (r, 0, 0)),
        ],
        out_specs=pl.BlockSpec((BN, H), lambda i, r: (r * NB + i, 0)),
        out_shape=jax.ShapeDtypeStruct((RN, H), jnp.float32),
    )(h1p, b1, W2)


def _run_chunk_pipeline(tab_ref, g1_v, b1_v, d2_v, rows_a, rows_b, acc,
                        sem_ga, sem_gb, sem_sa, sem_sb, nq):
    def gfire(r, buf, sem):
        pltpu.async_copy(tab_ref.at[g1_v.at[pl.ds(r * BT, BT)]], buf, sem)

    def gwait(buf, sem):
        pltpu.make_async_copy(
            tab_ref.at[g1_v.at[pl.ds(0, BT)]], buf, sem).wait()

    def swait(buf, sem):
        pltpu.make_async_copy(buf, acc.at[d2_v.at[0]], sem).wait()

    def batch(r, buf, gsem, ssem, obuf, ogsem, ossem):
        @pl.when(r > 0)
        def _():
            swait(obuf, ossem)

        @pl.when(r < ROWS - 1)
        def _():
            gfire(r + 1, obuf, ogsem)

        gwait(buf, gsem)

        def scale(g, _):
            c16 = b1_v[pl.ds(r * BT + g * 16, 16)]
            for j in range(16):
                cv = c16[j]
                e = g * 16 + j
                for q2 in range(nq):
                    buf[e, pl.ds(q2 * 16, 16)] = \
                        buf[e, pl.ds(q2 * 16, 16)] * cv
            return 0
        lax.fori_loop(0, BT // 16, scale, 0)
        pltpu.async_copy(buf, acc.at[d2_v.at[r]], ssem, add=True)

    gfire(0, rows_a, sem_ga)

    def step(r, _):
        even = lax.rem(r, 2) == 0

        @pl.when(even)
        def _():
            batch(r, rows_a, sem_ga, sem_sa, rows_b, sem_gb, sem_sb)

        @pl.when(jnp.logical_not(even))
        def _():
            batch(r, rows_b, sem_gb, sem_sb, rows_a, sem_ga, sem_sa)
        return 0
    lax.fori_loop(0, ROWS, step, 0)
    swait(rows_a, sem_sa)


def _pass1_body(tab_ref, gidx_ref, kdst_ref, edst_ref, nsrc_ref, ndst_ref,
                zr_ref,
                h1p_ref, b_ref,
                g1_v, d1_v, k1_v, b1_v, n1_v, d2_v, rows_a, rows_b, acc,
                sem_ga, sem_gb, sem_sa, sem_sb, sem_b):
    c = lax.axis_index("c")
    s = lax.axis_index("s")
    wid = s * NC + c
    nslc = N // 10
    ebase = wid * ET

    @pl.when(s < 10)
    def _zero():
        pltpu.sync_copy(zr_ref, acc.at[pl.ds(s * nslc, nslc)])
    plsc.subcore_barrier()

    def chunk(j, _):
        cbase = ebase + j * CH
        pltpu.sync_copy(gidx_ref.at[pl.ds(cbase, CH)], g1_v)
        pltpu.sync_copy(edst_ref.at[pl.ds(cbase, CH)], d1_v)
        pltpu.sync_copy(kdst_ref.at[pl.ds(cbase, CH)], k1_v)

        def repack(k, _):
            r = k // (BT // 16)
            q = (k % (BT // 16)) * 16
            d2_v[r, pl.ds(q, 16)] = d1_v[pl.ds(k * 16, 16)]
            return 0
        lax.fori_loop(0, CH // 16, repack, 0)

        def bfire(r, _):
            pltpu.async_copy(ndst_ref.at[k1_v.at[pl.ds(r * BT, BT)]],
                             b1_v.at[pl.ds(r * BT, BT)], sem_b)
            pltpu.async_copy(nsrc_ref.at[g1_v.at[pl.ds(r * BT, BT)]],
                             n1_v.at[pl.ds(r * BT, BT)], sem_b)
            return 0
        lax.fori_loop(0, ROWS, bfire, 0)

        def bdrain(r, _):
            pltpu.make_async_copy(ndst_ref.at[k1_v.at[pl.ds(0, BT)]],
                                  b1_v.at[pl.ds(0, BT)], sem_b).wait()
            pltpu.make_async_copy(ndst_ref.at[k1_v.at[pl.ds(0, BT)]],
                                  n1_v.at[pl.ds(0, BT)], sem_b).wait()
            return 0
        lax.fori_loop(0, ROWS, bdrain, 0)

        def combine(k, _):
            b1_v[pl.ds(k * 16, 16)] = (b1_v[pl.ds(k * 16, 16)]
                                       * n1_v[pl.ds(k * 16, 16)])
            return 0
        lax.fori_loop(0, CH // 16, combine, 0)
        pltpu.sync_copy(b1_v, b_ref.at[pl.ds(cbase, CH)])

        _run_chunk_pipeline(tab_ref, g1_v, b1_v, d2_v, rows_a, rows_b, acc,
                            sem_ga, sem_gb, sem_sa, sem_sb, H // 16)
        return 0
    lax.fori_loop(0, NCH, chunk, 0)
    plsc.subcore_barrier()

    @pl.when(s < 10)
    def _export():
        pltpu.sync_copy(acc.at[pl.ds(s * nslc, nslc)],
                        h1p_ref.at[c, pl.ds(s * nslc, nslc)])


_edge_pass1 = pl.kernel(
    _pass1_body,
    out_type=(
        jax.ShapeDtypeStruct((NC, N, H), jnp.float32),
        jax.ShapeDtypeStruct((E,), jnp.float32),
    ),
    mesh=_sc_mesh,
    scratch_types=[
        pltpu.VMEM((CH,), jnp.int32),
        pltpu.VMEM((CH,), jnp.int32),
        pltpu.VMEM((CH,), jnp.int32),
        pltpu.VMEM((CH,), jnp.float32),
        pltpu.VMEM((CH,), jnp.float32),
        pltpu.VMEM((ROWS, BT), jnp.int32),
        pltpu.VMEM((BT, H), jnp.float32),
        pltpu.VMEM((BT, H), jnp.float32),
        pltpu.VMEM_SHARED((N, H), jnp.float32),
        pltpu.SemaphoreType.DMA,
        pltpu.SemaphoreType.DMA,
        pltpu.SemaphoreType.DMA,
        pltpu.SemaphoreType.DMA,
        pltpu.SemaphoreType.DMA,
    ],
)


def _pass2_body(tab_ref, gidx_ref, edst_ref, b_ref, zr_ref,
                h2p_ref,
                g1_v, d1_v, b1_v, d2_v, rows_a, rows_b, acc,
                sem_ga, sem_gb, sem_sa, sem_sb):
    c = lax.axis_index("c")
    s = lax.axis_index("s")
    wid = s * NC + c
    nslc = N // 10
    ebase = wid * ET

    @pl.when(s < 10)
    def _zero():
        pltpu.sync_copy(zr_ref, acc.at[pl.ds(s * nslc, nslc)])
    plsc.subcore_barrier()

    def chunk(j, _):
        cbase = ebase + j * CH
        pltpu.sync_copy(gidx_ref.at[pl.ds(cbase, CH)], g1_v)
        pltpu.sync_copy(edst_ref.at[pl.ds(cbase, CH)], d1_v)
        pltpu.sync_copy(b_ref.at[pl.ds(cbase, CH)], b1_v)

        def repack(k, _):
            r = k // (BT // 16)
            q = (k % (BT // 16)) * 16
            d2_v[r, pl.ds(q, 16)] = d1_v[pl.ds(k * 16, 16)]
            return 0
        lax.fori_loop(0, CH // 16, repack, 0)

        _run_chunk_pipeline(tab_ref, g1_v, b1_v, d2_v, rows_a, rows_b, acc,
                            sem_ga, sem_gb, sem_sa, sem_sb, O // 16)
        return 0
    lax.fori_loop(0, NCH, chunk, 0)
    plsc.subcore_barrier()

    @pl.when(s < 10)
    def _export():
        pltpu.sync_copy(acc.at[pl.ds(s * nslc, nslc)],
                        h2p_ref.at[c, pl.ds(s * nslc, nslc)])


_edge_pass2 = pl.kernel(
    _pass2_body,
    out_type=jax.ShapeDtypeStruct((NC, N, H), jnp.float32),
    mesh=_sc_mesh,
    scratch_types=[
        pltpu.VMEM((CH,), jnp.int32),
        pltpu.VMEM((CH,), jnp.int32),
        pltpu.VMEM((CH,), jnp.float32),
        pltpu.VMEM((ROWS, BT), jnp.int32),
        pltpu.VMEM((BT, H), jnp.float32),
        pltpu.VMEM((BT, H), jnp.float32),
        pltpu.VMEM_SHARED((N, H), jnp.float32),
        pltpu.SemaphoreType.DMA,
        pltpu.SemaphoreType.DMA,
        pltpu.SemaphoreType.DMA,
        pltpu.SemaphoreType.DMA,
    ],
)


def _final_body(hp_ref, b2_ref, wf_ref, bf_ref, wb_ref, bb_ref, o_ref):
    mb = jnp.mean(b2_ref[...], axis=0)
    h = (hp_ref[0, :, :O] + hp_ref[1, :, :O]) * (1.0 / R) + mb[None, :]

    def lstm(w, bias):
        gates = jnp.dot(h, w, preferred_element_type=jnp.float32) + bias
        hh = O // 2
        i_ = gates[:, :hh]
        g_ = gates[:, 2 * hh:3 * hh]
        o_ = gates[:, 3 * hh:]
        cc = jax.nn.sigmoid(i_) * jnp.tanh(g_)
        return jax.nn.sigmoid(o_) * jnp.tanh(cc)

    o_ref[...] = jnp.concatenate(
        [lstm(wf_ref[...], bf_ref[...]), lstm(wb_ref[...], bb_ref[...])],
        axis=1)


def _final(h2p, b2, WihT_f, bias_f, WihT_b, bias_b):
    return pl.pallas_call(
        _final_body,
        grid=(NB,),
        in_specs=[
            pl.BlockSpec((2, BN, H), lambda i: (0, i, 0)),
            pl.BlockSpec((R, O), lambda i: (0, 0)),
            pl.BlockSpec((O, 2 * O), lambda i: (0, 0)),
            pl.BlockSpec((1, 2 * O), lambda i: (0, 0)),
            pl.BlockSpec((O, 2 * O), lambda i: (0, 0)),
            pl.BlockSpec((1, 2 * O), lambda i: (0, 0)),
        ],
        out_specs=pl.BlockSpec((BN, O), lambda i: (i, 0)),
        out_shape=jax.ShapeDtypeStruct((N, O), jnp.float32),
    )(h2p, b2, WihT_f, bias_f, WihT_b, bias_b)


def kernel(node_ids, edge_index, edge_type, entity_emb, W1, b1, W2, b2,
           Wih_f, bih_f, bhh_f, Wih_b, bih_b, bhh_b):
    x = jnp.take(entity_emb, node_ids, axis=0)
    z_deg = jnp.zeros((ZCH,), jnp.float32)
    zr = jnp.zeros((N // 10, H), jnp.float32)

    esrc = edge_index[0]
    edst = edge_index[1]
    gidx, kdst, degp = _edge_prep(esrc, edst, edge_type, z_deg)
    norms = _norms(degp).reshape(2, PAD)
    nsrc1d = norms[0, :RN]
    ndst1d = norms[1, :RN]

    table1 = _table1(x, W1)
    h1p, c_e = _edge_pass1(table1, gidx, kdst, edst, nsrc1d, ndst1d, zr)
    table2 = _table2(h1p, b1, W2)
    h2p = _edge_pass2(table2, gidx, edst, c_e, zr)

    bias_f = (bih_f + bhh_f).reshape(1, 2 * O)
    bias_b = (bih_b + bhh_b).reshape(1, 2 * O)
    return _final(h2p, b2, Wih_f.T, bias_f, Wih_b.T, bias_b)
